# Initial kernel scaffold; baseline (speedup 1.0000x reference)
import jax
import jax.numpy as jnp
from jax.experimental import pallas as pl


def kernel(x_gene, x_patient, x_group, ei_g2p, ei_p2g, ei_g2m, ei_m2g, params):
    raise NotImplementedError("write your pallas kernel here")


# speedup vs baseline: 4.4181x; 4.4181x over previous
import functools

import jax
import jax.numpy as jnp
from jax import lax
from jax.experimental import pallas as pl
from jax.experimental.pallas import tpu as pltpu
from jax.experimental.pallas import tpu_sc as plsc

F32 = jnp.float32
HID = 64
NC, NS = 2, 16
LANES = 16
NCHUNK = 4

N_GENE, N_PAT, N_GRP = 100000, 10000, 1000
P_GENE, P_PAT, P_GRP = 100352, 10240, 1024

ZROWS = 1024
ZW1 = 1568
RBUF = 8

ECFG = [
    dict(name="g2p", src="gene", nsrc=N_GENE, ndst=N_PAT, npad=P_PAT,
         e=500000, epad=507904, blk=8, stage=False),
    dict(name="p2g", src="patient", nsrc=N_PAT, ndst=N_GENE, npad=P_GENE,
         e=500000, epad=507904, blk=8, stage=False),
    dict(name="g2m", src="gene", nsrc=N_GENE, ndst=N_GRP, npad=P_GRP,
         e=100000, epad=114688, blk=8, stage=False),
    dict(name="m2g", src="group", nsrc=N_GRP, ndst=N_GENE, npad=P_GENE,
         e=100000, epad=114688, blk=8, stage=True),
]
BLKMAX = 8

@functools.cache
def _mesh():
    return plsc.VectorSubcoreMesh(core_axis_name="c", subcore_axis_name="s",
                                  num_cores=NC, num_subcores=NS)


def _zero_acc(zv, acc, rpt, off, zrows):
    n_full, rem = divmod(rpt, zrows)
    for t in range(n_full):
        pltpu.sync_copy(zv, acc.at[pl.ds(off + t * zrows, zrows)])
    if rem:
        pltpu.sync_copy(zv.at[pl.ds(0, rem)],
                        acc.at[pl.ds(off + n_full * zrows, rem)])


def _segsum_body(h4_gene, h4_pat, h4_grp,
                 si0, di0, si1, di1, si2, di2, si3, di3, zhbm,
                 o0, o1, o2, o3,
                 sidx_v, didx_v, rows_v, acc, stage, gsem, ssem):
    c = lax.axis_index("c")
    s = lax.axis_index("s")
    srcs = {"gene": h4_gene, "patient": h4_pat, "group": h4_grp}
    sis = [si0, si1, si2, si3]
    dis = [di0, di1, di2, di3]
    outs = [o0, o1, o2, o3]
    for cfg, si, di, out in zip(ECFG, sis, dis, outs):
        src = srcs[cfg["src"]]
        blk = cfg["blk"]
        rpt = cfg["npad"] // NS
        off = s * rpt
        nstr = (cfg["epad"] // 128) // NS
        base = s * nstr
        for k in range(2):
            chunk = c * 2 + k
            pltpu.sync_copy(zhbm, rows_v.at[pl.ds(0, ZROWS)])
            _zero_acc(rows_v.at[pl.ds(0, ZROWS)], acc, rpt, off, ZROWS)
            if cfg["stage"]:
                @pl.when(s == 0)
                def _stage():
                    pltpu.sync_copy(src.at[chunk, pl.ds(0, cfg["nsrc"])],
                                    stage.at[pl.ds(0, cfg["nsrc"])])
            plsc.subcore_barrier()

            def blk_body(b, carry):
                r0 = base + b * blk
                pltpu.sync_copy(si.at[pl.ds(r0, blk)], sidx_v.at[pl.ds(0, blk)])
                pltpu.sync_copy(di.at[pl.ds(r0, blk)], didx_v.at[pl.ds(0, blk)])
                gdescs = []
                for q in range(blk):
                    if cfg["stage"]:
                        sref = stage.at[sidx_v.at[q]]
                    else:
                        sref = src.at[chunk].at[sidx_v.at[q]]
                    gdescs.append(pltpu.async_copy(
                        sref, rows_v.at[pl.ds(q * 128, 128)], gsem))
                sdescs = []
                for q in range(blk):
                    gdescs[q].wait()
                    sdescs.append(pltpu.async_copy(
                        rows_v.at[pl.ds(q * 128, 128)],
                        acc.at[didx_v.at[q]], ssem, add=True))
                for q in range(blk):
                    sdescs[q].wait()
                return carry

            lax.fori_loop(0, nstr // blk, blk_body, 0, unroll=False)
            plsc.subcore_barrier()
            col = chunk * LANES
            zb = rows_v.at[pl.ds(0, ZROWS)]
            n_full, rem = divmod(rpt, ZROWS)
            for t in range(n_full):
                o2_ = off + t * ZROWS
                pltpu.sync_copy(acc.at[pl.ds(o2_, ZROWS)], zb)
                pltpu.sync_copy(zb, out.at[pl.ds(o2_, ZROWS), pl.ds(col, LANES)])
            if rem:
                o2_ = off + n_full * ZROWS
                pltpu.sync_copy(acc.at[pl.ds(o2_, rem)],
                                rows_v.at[pl.ds(0, rem)])
                pltpu.sync_copy(rows_v.at[pl.ds(0, rem)],
                                out.at[pl.ds(o2_, rem), pl.ds(col, LANES)])
            plsc.subcore_barrier()


@functools.cache
def _segsum_kernel():
    return pl.kernel(
        _segsum_body,
        out_type=[jax.ShapeDtypeStruct((P_PAT, HID), F32),
                  jax.ShapeDtypeStruct((P_GENE, HID), F32),
                  jax.ShapeDtypeStruct((P_GRP, HID), F32),
                  jax.ShapeDtypeStruct((P_GENE, HID), F32)],
        mesh=_mesh(),
        scratch_types=[
            pltpu.VMEM((BLKMAX, 128), jnp.int32),
            pltpu.VMEM((BLKMAX, 128), jnp.int32),
            pltpu.VMEM((RBUF * 128, LANES), F32),
            pltpu.VMEM_SHARED((P_GENE, LANES), F32),
            pltpu.VMEM_SHARED((P_GRP, LANES), F32),
            pltpu.SemaphoreType.DMA,
            pltpu.SemaphoreType.DMA,
        ],
        compiler_params=pltpu.CompilerParams(use_tc_tiling_on_sc=False),
    )


def _segsum(*args):
    return _segsum_kernel()(*args)


def _counts_body(di0, di1, di2, di3, ones_hbm, z1_hbm,
                 o0, o1, o2, o3,
                 didx_v, ones_v, zv1, acc, ssem):
    c = lax.axis_index("c")
    s = lax.axis_index("s")
    pltpu.sync_copy(ones_hbm, ones_v)
    pltpu.sync_copy(z1_hbm, zv1)
    dis = [di0, di1, di2, di3]
    outs = [o0, o1, o2, o3]
    for cfg, di, out in zip(ECFG, dis, outs):
        blk = cfg["blk"]
        rpt = cfg["npad"] // NS
        off = s * rpt
        nstr = (cfg["epad"] // 128) // NS
        hi = ((nstr + 15) // 16) * 8
        start = s * nstr + c * hi
        nblk = jnp.where(c == 0, hi // blk, (nstr - hi) // blk)
        _zero_acc(zv1, acc, rpt, off, ZW1)
        plsc.subcore_barrier()

        def blk_body(b, carry):
            r0 = start + b * blk
            pltpu.sync_copy(di.at[pl.ds(r0, blk)], didx_v.at[pl.ds(0, blk)])
            sdescs = []
            for j in range(blk):
                sdescs.append(pltpu.async_copy(
                    ones_v, acc.at[didx_v.at[j]], ssem, add=True))
            for j in range(blk):
                sdescs[j].wait()
            return carry

        lax.fori_loop(0, nblk, blk_body, 0, unroll=False)
        plsc.subcore_barrier()
        n_full, rem = divmod(rpt, ZW1)
        for t in range(n_full):
            o2_ = off + t * ZW1
            pltpu.sync_copy(acc.at[pl.ds(o2_, ZW1)], zv1)
            pltpu.sync_copy(zv1, out.at[c, pl.ds(o2_, ZW1)])
        if rem:
            o2_ = off + n_full * ZW1
            pltpu.sync_copy(acc.at[pl.ds(o2_, rem)], zv1.at[pl.ds(0, rem)])
            pltpu.sync_copy(zv1.at[pl.ds(0, rem)],
                            out.at[c, pl.ds(o2_, rem)])
        pltpu.sync_copy(z1_hbm, zv1)
        plsc.subcore_barrier()


@functools.cache
def _counts_kernel():
    return pl.kernel(
        _counts_body,
        out_type=[jax.ShapeDtypeStruct((NC, P_PAT), F32),
                  jax.ShapeDtypeStruct((NC, P_GENE), F32),
                  jax.ShapeDtypeStruct((NC, P_GRP), F32),
                  jax.ShapeDtypeStruct((NC, P_GENE), F32)],
        mesh=_mesh(),
        scratch_types=[
            pltpu.VMEM((BLKMAX, 128), jnp.int32),
            pltpu.VMEM((128,), F32),
            pltpu.VMEM((ZW1,), F32),
            pltpu.VMEM_SHARED((P_GENE,), F32),
            pltpu.SemaphoreType.DMA,
        ],
        compiler_params=pltpu.CompilerParams(use_tc_tiling_on_sc=False),
    )


def _counts(*args):
    return _counts_kernel()(*args)


def _elu(x):
    return jnp.where(x > 0, x, jnp.exp(x) - 1.0)


def _enc_kernel(x_ref, wt_ref, b_ref, o_ref, o4_ref):
    h = jnp.dot(x_ref[...], wt_ref[...], preferred_element_type=F32) + b_ref[...]
    h = _elu(h)
    o_ref[...] = h
    o4_ref[...] = _split4(h)


def _encode(x, wt, b, block):
    n = x.shape[0]
    return pl.pallas_call(
        _enc_kernel,
        grid=(n // block,),
        in_specs=[pl.BlockSpec((block, 16), lambda i: (i, 0)),
                  pl.BlockSpec((16, HID), lambda i: (0, 0)),
                  pl.BlockSpec((1, HID), lambda i: (0, 0))],
        out_specs=[pl.BlockSpec((block, HID), lambda i: (i, 0)),
                   pl.BlockSpec((NCHUNK, block, LANES), lambda i: (0, i, 0))],
        out_shape=[jax.ShapeDtypeStruct((n, HID), F32),
                   jax.ShapeDtypeStruct((NCHUNK, n, LANES), F32)],
    )(x, wt, b)


def _cat4(ref):
    return jnp.concatenate([ref[i] for i in range(NCHUNK)], axis=-1)


def _split4(x):
    return jnp.stack([x[:, 16 * i:16 * (i + 1)] for i in range(NCHUNK)], axis=0)


def _make_combine_kernel(n_in, want_h4):
    def kern(*refs):
        h = refs[0][...]
        pos = 1
        acc = jnp.zeros_like(h)
        for _ in range(n_in):
            s_ref, cnt, wlT = refs[pos], refs[pos + 1], refs[pos + 2]
            pos += 3
            inv = 1.0 / jnp.maximum(cnt[...], 1.0)
            mean = s_ref[...] * jnp.broadcast_to(inv, h.shape)
            acc = acc + jnp.dot(mean, wlT[...], preferred_element_type=F32)
        wrT, bsum, lnw, lnb = refs[pos:pos + 4]
        pos += 4
        acc = acc + bsum[...] + jnp.dot(h, wrT[...], preferred_element_type=F32)
        z = h + acc
        mu = jnp.mean(z, axis=-1, keepdims=True)
        zc = z - mu
        var = jnp.mean(zc * zc, axis=-1, keepdims=True)
        zn = zc * lax.rsqrt(var + 1e-5) * lnw[...] + lnb[...]
        hn = _elu(zn)
        refs[pos][...] = hn
        if want_h4:
            refs[pos + 1][...] = _split4(hn)
    return kern


def _combine(h64, ins, wrT, bsum, lnw, lnb, block, want_h4):
    n_in = len(ins)
    n = h64.shape[0]
    grid = (n // block,)
    in_specs = [pl.BlockSpec((block, HID), lambda i: (i, 0))]
    args = [h64]
    for (s, cnt, wlT) in ins:
        in_specs += [pl.BlockSpec((block, HID), lambda i: (i, 0)),
                     pl.BlockSpec((block, 1), lambda i: (i, 0)),
                     pl.BlockSpec((HID, HID), lambda i: (0, 0))]
        args += [s, cnt, wlT]
    in_specs += [pl.BlockSpec((HID, HID), lambda i: (0, 0)),
                 pl.BlockSpec((1, HID), lambda i: (0, 0)),
                 pl.BlockSpec((1, HID), lambda i: (0, 0)),
                 pl.BlockSpec((1, HID), lambda i: (0, 0))]
    args += [wrT, bsum, lnw, lnb]
    out_specs = [pl.BlockSpec((block, HID), lambda i: (i, 0))]
    out_shape = [jax.ShapeDtypeStruct((n, HID), F32)]
    if want_h4:
        out_specs.append(pl.BlockSpec((NCHUNK, block, LANES),
                                      lambda i: (0, i, 0)))
        out_shape.append(jax.ShapeDtypeStruct((NCHUNK, n, LANES), F32))
    res = pl.pallas_call(
        _make_combine_kernel(n_in, want_h4),
        grid=grid,
        in_specs=in_specs,
        out_specs=out_specs,
        out_shape=out_shape,
    )(*args)
    return res if want_h4 else (res[0], None)


def _head_kernel(h_ref, w1t_ref, b1_ref, w2t_ref, b2_ref, o_ref):
    h = h_ref[...]
    hid = _elu(jnp.dot(h, w1t_ref[...], preferred_element_type=F32) + b1_ref[...])
    o_ref[...] = jnp.dot(hid, w2t_ref[...], preferred_element_type=F32) + b2_ref[...]


def _head(h64, w1t, b1, w2t8, b2_8, block):
    n = h64.shape[0]
    return pl.pallas_call(
        _head_kernel,
        grid=(n // block,),
        in_specs=[pl.BlockSpec((block, HID), lambda i: (i, 0)),
                  pl.BlockSpec((HID, 32), lambda i: (0, 0)),
                  pl.BlockSpec((1, 32), lambda i: (0, 0)),
                  pl.BlockSpec((32, 8), lambda i: (0, 0)),
                  pl.BlockSpec((1, 8), lambda i: (0, 0))],
        out_specs=pl.BlockSpec((block, 8), lambda i: (i, 0)),
        out_shape=jax.ShapeDtypeStruct((n, 8), F32),
    )(h64, w1t, b1, w2t8, b2_8)


def _prep_edges(ei, epad, ndst):
    src = ei[0].astype(jnp.int32)
    dst = ei[1].astype(jnp.int32)
    padn = epad - src.shape[0]
    fill = jnp.arange(padn, dtype=jnp.int32)
    src = jnp.concatenate([src, fill % 64])
    dst = jnp.concatenate([dst, ndst + (fill % 64)])
    return src.reshape(-1, 128), dst.reshape(-1, 128)


def kernel(x_gene, x_patient, x_group, ei_g2p, ei_p2g, ei_g2m, ei_m2g, params):
    p = params

    xg = jnp.pad(x_gene, ((0, 0), (0, 16 - x_gene.shape[1])))
    xp = jnp.pad(x_patient, ((0, 0), (0, 16 - x_patient.shape[1])))
    xm = jnp.pad(x_group, ((0, 0), (0, 16 - x_group.shape[1])))
    wgT = jnp.pad(p["W_gene"].T, ((0, 16 - x_gene.shape[1]), (0, 0)))
    wpT = jnp.pad(p["W_patient"].T, ((0, 16 - x_patient.shape[1]), (0, 0)))
    wmT = jnp.pad(p["W_group"].T, ((0, 16 - x_group.shape[1]), (0, 0)))

    eidx = {}
    for cfg, ei in zip(ECFG, [ei_g2p, ei_p2g, ei_g2m, ei_m2g]):
        eidx[cfg["name"]] = _prep_edges(ei, cfg["epad"], cfg["ndst"])

    zhbm = jnp.zeros((ZROWS, LANES), F32)
    ones128 = jnp.ones((128,), F32)
    z1 = jnp.zeros((ZW1,), F32)

    hg, h4g = _encode(xg, wgT, p["b_gene"].reshape(1, HID), 2000)
    hp, h4p = _encode(xp, wpT, p["b_patient"].reshape(1, HID), 2000)
    hm, h4m = _encode(xm, wmT, p["b_group"].reshape(1, HID), 1000)

    cnt_g2p, cnt_p2g, cnt_g2m, cnt_m2g = _counts(
        eidx["g2p"][1], eidx["p2g"][1], eidx["g2m"][1], eidx["m2g"][1],
        ones128, z1)
    cnt = {"g2p": (cnt_g2p[0] + cnt_g2p[1]).reshape(-1, 1),
           "p2g": (cnt_p2g[0] + cnt_p2g[1]).reshape(-1, 1),
           "g2m": (cnt_g2m[0] + cnt_g2m[1]).reshape(-1, 1),
           "m2g": (cnt_m2g[0] + cnt_m2g[1]).reshape(-1, 1)}

    for l in range(2):
        s_g2p, s_p2g, s_g2m, s_m2g = _segsum(
            h4g, h4p, h4m,
            eidx["g2p"][0], eidx["g2p"][1],
            eidx["p2g"][0], eidx["p2g"][1],
            eidx["g2m"][0], eidx["g2m"][1],
            eidx["m2g"][0], eidx["m2g"][1],
            zhbm)

        def wlT(name):
            return p[f"Wl_{l}_{name}"].T

        def wrT(*names):
            w = sum(p[f"Wr_{l}_{n}"] for n in names)
            return w.T

        def lnw(nt):
            return p[f"ln_w_{l}_{nt}"].reshape(1, HID)

        def lnb(nt):
            return p[f"ln_b_{l}_{nt}"].reshape(1, HID)

        want_h4 = l == 0
        hp, h4p = _combine(
            hp, [(s_g2p, cnt["g2p"], wlT("g2p"))],
            wrT("g2p"), p[f"bl_{l}_g2p"].reshape(1, HID),
            lnw("patient"), lnb("patient"), 2000, want_h4)
        hg, h4g = _combine(
            hg, [(s_p2g, cnt["p2g"], wlT("p2g")),
                 (s_m2g, cnt["m2g"], wlT("m2g"))],
            wrT("p2g", "m2g"),
            (p[f"bl_{l}_p2g"] + p[f"bl_{l}_m2g"]).reshape(1, HID),
            lnw("gene"), lnb("gene"), 2000, want_h4)
        hm, h4m = _combine(
            hm, [(s_g2m, cnt["g2m"], wlT("g2m"))],
            wrT("g2m"), p[f"bl_{l}_g2m"].reshape(1, HID),
            lnw("mutation_group"), lnb("mutation_group"), 1000, want_h4)

    w2t8 = jnp.tile(p["W_c2"].T, (1, 8))
    b2_8 = jnp.tile(p["b_c2"].reshape(1, 1), (1, 8))
    out8 = _head(hp, p["W_c1"].T, p["b_c1"].reshape(1, 32), w2t8, b2_8, 2000)
    return out8[:, 0]
